# baseline (speedup 1.0000x reference)
import functools

import jax
import jax.numpy as jnp
from jax import lax
from jax.experimental import pallas as pl
from jax.experimental.pallas import tpu as pltpu
from jax.experimental.pallas import tpu_sc as plsc

_L = 16


def _i0():
    return jnp.int32(0)


def _iota16():
    return lax.iota(jnp.int32, 16)


def _splat(x):
    return jnp.full((_L,), x, dtype=jnp.int32)


def _cummin(v):
    return -plsc.cummax(-v)


def _rev(v):
    return lax.rev(v, (0,))


def _sc_gamma_body(Tt, Ts, scores_hbm, gamma_hbm, read_hbm,
                   sc_v, col_v, fs_v, cs_v, best_v, gam_v, read_v):
    B = scores_hbm.shape[0]
    NC = 2
    nk = Ts // _L
    wid = lax.axis_index("s") * NC + lax.axis_index("c")

    @pl.when(wid < B)
    def _worker():
        pltpu.sync_copy(scores_hbm.at[wid], sc_v)

        iot = _iota16()
        fiot = [(_iota16() + k * _L).astype(jnp.float32) for k in range(nk)]
        R = [fiot[k] * (fiot[k] + 3.0) * (0.5 / Ts) for k in range(nk)]
        C = [fiot[k] * (fiot[k] + 1.0) * (0.5 / Ts) for k in range(nk)]
        row0 = [(fiot[k] + 1.0) * (fiot[k] + 2.0) * (0.5 / Ts) for k in range(nk)]
        rowT = [(Ts - fiot[k]) * (1.0 / Ts) for k in range(nk)]

        zer = _splat(0)
        for col, slot in ((0, 0), (Ts - 1, 1)):
            carry = jnp.float32(0.0)
            for k in range(Tt // _L):
                rows = iot + k * _L
                g = plsc.load_gather(sc_v, [rows, _splat(col)])
                x = -g
                col_v[slot, pl.ds(k * _L, _L)] = plsc.cumsum(x) + carry
                carry = carry + jnp.sum(x)

        for k in range(nk):
            fs_v[0, pl.ds(k * _L, _L)] = row0[k]

        def fwd_step(t, fs_prev):
            c0 = plsc.load_gather(col_v, [zer, _splat(t)])
            u = []
            for k in range(nk):
                srow = sc_v[t, pl.ds(k * _L, _L)]
                uk = fs_prev[k] - srow - R[k]
                if k == 0:
                    uk = jnp.where(iot == 0, c0, uk)
                u.append(uk)
            out = []
            carry = jnp.float32(jnp.inf)
            for k in range(nk):
                run = jnp.minimum(_cummin(u[k]), carry)
                carry = run[_L - 1]
                fk = run + R[k]
                fs_v[t + 1, pl.ds(k * _L, _L)] = fk
                out.append(fk)
            return tuple(out)

        lax.fori_loop(jnp.int32(0), jnp.int32(Tt), fwd_step, tuple(row0), unroll=False)

        for k in range(nk):
            cs_v[Tt, pl.ds(k * _L, _L)] = fs_v[Tt, pl.ds(k * _L, _L)] + rowT[k]

        def bwd_step(i, bs_next):
            t = Tt - 1 - i
            cL = plsc.load_gather(col_v, [_splat(1), _splat(Tt - 1 - t)])
            w = []
            for k in range(nk):
                srow = sc_v[t, pl.ds(k * _L, _L)]
                wk = bs_next[k] - srow + C[k]
                if k == nk - 1:
                    wk = jnp.where(iot == _L - 1, cL + C[nk - 1], wk)
                w.append(wk)
            out = []
            carry = jnp.float32(jnp.inf)
            for k in range(nk - 1, -1, -1):
                sfx = jnp.minimum(_rev(_cummin(_rev(w[k]))), carry)
                carry = sfx[0]
                bk = sfx - C[k]
                fk = fs_v[t, pl.ds(k * _L, _L)]
                cs_v[t, pl.ds(k * _L, _L)] = bk + fk
                out.append(bk)
            out.reverse()
            return tuple(out)

        lax.fori_loop(jnp.int32(0), jnp.int32(Tt), bwd_step, tuple(rowT), unroll=False)

        for k in range(Tt // _L):
            best_v[pl.ds(k * _L, _L)] = _splat(Ts - 1)
        lane0 = iot == 0

        def walk_step(_, tj):
            tv, jv = tj
            active = (tv < Tt) & (jv < Ts - 1)
            tc = jnp.minimum(tv, Tt - 1)
            jc = jnp.minimum(jv, Ts - 2)
            g_dn = plsc.load_gather(cs_v, [tc + 1, jc])
            g_rt = plsc.load_gather(cs_v, [tc, jc + 1])
            do_write = active & (g_dn <= g_rt)
            do_read = active & jnp.logical_not(do_write)
            plsc.store_scatter(best_v, [tc], jc, mask=do_write & lane0)
            tv = jnp.where(do_write, tv + 1, tv)
            jv = jnp.where(do_read, jv + 1, jv)
            return (tv, jv)

        lax.fori_loop(jnp.int32(0), jnp.int32(Tt + Ts - 1), walk_step, (_splat(0), _splat(0)),
                      unroll=False)

        def gamma_step(t, c):
            bvec = plsc.load_gather(best_v, [_splat(t)])
            for k in range(nk):
                gk = jnp.where(iot + k * _L >= bvec,
                               jnp.float32(1.0), jnp.float32(0.0))
                gam_v[t, pl.ds(k * _L, _L)] = gk
            return c

        lax.fori_loop(jnp.int32(0), jnp.int32(Tt), gamma_step, jnp.int32(0), unroll=False)

        def read_step(t, c):
            for k in range(nk):
                gk = gam_v[t + 1, pl.ds(k * _L, _L)]
                read_v[t, pl.ds(k * _L, _L)] = 1.0 - gk
            return c

        lax.fori_loop(jnp.int32(0), jnp.int32(Tt - 1), read_step, jnp.int32(0), unroll=False)

        pltpu.sync_copy(gam_v, gamma_hbm.at[wid])
        pltpu.sync_copy(read_v, read_hbm.at[wid])


def _sc_gamma(scores):
    B, Tt, Ts = scores.shape
    mesh = plsc.VectorSubcoreMesh(core_axis_name="c", subcore_axis_name="s")
    kern = pl.kernel(
        functools.partial(_sc_gamma_body, Tt, Ts),
        out_type=[
            jax.ShapeDtypeStruct((B, Tt, Ts), jnp.float32),
            jax.ShapeDtypeStruct((B, Tt - 1, Ts), jnp.float32),
        ],
        mesh=mesh,
        compiler_params=pltpu.CompilerParams(needs_layout_passes=False),
        scratch_types=[
            pltpu.VMEM((Tt, Ts), jnp.float32),
            pltpu.VMEM((2, Tt), jnp.float32),
            pltpu.VMEM((Tt + 1, Ts), jnp.float32),
            pltpu.VMEM((Tt + 1, Ts), jnp.float32),
            pltpu.VMEM((Tt,), jnp.int32),
            pltpu.VMEM((Tt, Ts), jnp.float32),
            pltpu.VMEM((Tt - 1, Ts), jnp.float32),
        ],
    )
    return kern(scores)


def _tc_gate_body(obs_ref, w_ref, b_ref, s_ref, sx_ref):
    x = jax.lax.dot_general(
        obs_ref[...], w_ref[...], (((1,), (0,)), ((), ())),
        preferred_element_type=jnp.float32,
    ) + b_ref[0, 0]
    s = jnp.minimum(x, 0.0) - jnp.log(1.0 + jnp.exp(-jnp.abs(x)))
    s_ref[...] = s
    sx_ref[...] = s - x


def _tc_gate(obs2, Wc, bc, rows_per_block):
    N, Cdim = obs2.shape
    grid = (N // rows_per_block,)
    return pl.pallas_call(
        _tc_gate_body,
        grid=grid,
        in_specs=[
            pl.BlockSpec((rows_per_block, Cdim), lambda i: (i, _i0())),
            pl.BlockSpec((Cdim, 1), lambda i: (_i0(), _i0())),
            pl.BlockSpec((8, 128), lambda i: (_i0(), _i0())),
        ],
        out_specs=[
            pl.BlockSpec((rows_per_block, 1), lambda i: (i, _i0())),
            pl.BlockSpec((rows_per_block, 1), lambda i: (i, _i0())),
        ],
        out_shape=[
            jax.ShapeDtypeStruct((N, 1), jnp.float32),
            jax.ShapeDtypeStruct((N, 1), jnp.float32),
        ],
    )(obs2, Wc, bc)


def kernel(observations, scores, W, b):
    B, Tt, Ts, Cdim = observations.shape
    obs2 = observations.reshape(B * Tt * Ts, Cdim)
    Wc = W.reshape(Cdim, 1)
    bc = jnp.broadcast_to(b.reshape(1, 1), (8, 128)).astype(jnp.float32)
    gamma, read = _sc_gamma(scores)
    s_flat, sx_flat = _tc_gate(obs2, Wc, bc, rows_per_block=4096)
    s3 = s_flat.reshape(B, Tt, Ts)[:, : Tt - 1]
    sx3 = sx_flat.reshape(B, Tt, Ts)[:, : Tt - 1]
    controls = jnp.stack([s3, sx3], axis=-1)
    write = gamma[:, 1:]
    return (controls, gamma, read, write)


# speedup vs baseline: 1.4088x; 1.4088x over previous
import functools

import jax
import jax.numpy as jnp
from jax import lax
from jax.experimental import pallas as pl
from jax.experimental.pallas import tpu as pltpu
from jax.experimental.pallas import tpu_sc as plsc

_L = 16


def _i0():
    return jnp.int32(0)


def _iota16():
    return lax.iota(jnp.int32, 16)


def _splat(x):
    return jnp.full((_L,), x, dtype=jnp.int32)


def _cummin(v):
    return -plsc.cummax(-v)


def _rev(v):
    return lax.rev(v, (0,))


def _sc_gamma_body(Tt, Ts, scores_hbm, gamma_hbm, read_hbm,
                   sc_v, col_v, fs_v, cs_v, best_v, gam_v, read_v):
    B = scores_hbm.shape[0]
    NC = 2
    nk = Ts // _L
    wid = lax.axis_index("s") * NC + lax.axis_index("c")

    @pl.when(wid < B)
    def _worker():
        pltpu.sync_copy(scores_hbm.at[wid], sc_v)

        iot = _iota16()
        fiot = [(_iota16() + k * _L).astype(jnp.float32) for k in range(nk)]
        R = [fiot[k] * (fiot[k] + 3.0) * (0.5 / Ts) for k in range(nk)]
        C = [fiot[k] * (fiot[k] + 1.0) * (0.5 / Ts) for k in range(nk)]
        row0 = [(fiot[k] + 1.0) * (fiot[k] + 2.0) * (0.5 / Ts) for k in range(nk)]
        rowT = [(Ts - fiot[k]) * (1.0 / Ts) for k in range(nk)]

        zer = _splat(0)
        for col, slot in ((0, 0), (Ts - 1, 1)):
            carry = jnp.float32(0.0)
            for k in range(Tt // _L):
                rows = iot + k * _L
                g = plsc.load_gather(sc_v, [rows, _splat(col)])
                x = -g
                col_v[slot, pl.ds(k * _L, _L)] = plsc.cumsum(x) + carry
                carry = carry + jnp.sum(x)

        for k in range(nk):
            fs_v[0, pl.ds(k * _L, _L)] = row0[k]

        def fwd_step(t, fs_prev):
            c0 = plsc.load_gather(col_v, [zer, _splat(t)])
            u = []
            for k in range(nk):
                srow = sc_v[t, pl.ds(k * _L, _L)]
                uk = fs_prev[k] - srow - R[k]
                if k == 0:
                    uk = jnp.where(iot == 0, c0, uk)
                u.append(uk)
            out = []
            carry = jnp.float32(jnp.inf)
            for k in range(nk):
                run = jnp.minimum(_cummin(u[k]), carry)
                carry = run[_L - 1]
                fk = run + R[k]
                fs_v[t + 1, pl.ds(k * _L, _L)] = fk
                out.append(fk)
            return tuple(out)

        lax.fori_loop(jnp.int32(0), jnp.int32(Tt), fwd_step, tuple(row0), unroll=False)

        for k in range(nk):
            cs_v[Tt, pl.ds(k * _L, _L)] = fs_v[Tt, pl.ds(k * _L, _L)] + rowT[k]

        def bwd_step(i, bs_next):
            t = Tt - 1 - i
            cL = plsc.load_gather(col_v, [_splat(1), _splat(Tt - 1 - t)])
            w = []
            for k in range(nk):
                srow = sc_v[t, pl.ds(k * _L, _L)]
                wk = bs_next[k] - srow + C[k]
                if k == nk - 1:
                    wk = jnp.where(iot == _L - 1, cL + C[nk - 1], wk)
                w.append(wk)
            out = []
            carry = jnp.float32(jnp.inf)
            for k in range(nk - 1, -1, -1):
                sfx = jnp.minimum(_rev(_cummin(_rev(w[k]))), carry)
                carry = sfx[0]
                bk = sfx - C[k]
                fk = fs_v[t, pl.ds(k * _L, _L)]
                cs_v[t, pl.ds(k * _L, _L)] = bk + fk
                out.append(bk)
            out.reverse()
            return tuple(out)

        lax.fori_loop(jnp.int32(0), jnp.int32(Tt), bwd_step, tuple(rowT), unroll=False)

        for k in range(Tt // _L):
            best_v[pl.ds(k * _L, _L)] = _splat(Ts - 1)
        lane0 = iot == 0

        def walk_step(_, tj):
            tv, jv = tj
            active = (tv < Tt) & (jv < Ts - 1)
            tc = jnp.minimum(tv, Tt - 1)
            jc = jnp.minimum(jv, Ts - 2)
            g_dn = plsc.load_gather(cs_v, [tc + 1, jc])
            g_rt = plsc.load_gather(cs_v, [tc, jc + 1])
            do_write = active & (g_dn <= g_rt)
            do_read = active & jnp.logical_not(do_write)
            plsc.store_scatter(best_v, [tc], jc, mask=do_write & lane0)
            tv = jnp.where(do_write, tv + 1, tv)
            jv = jnp.where(do_read, jv + 1, jv)
            return (tv, jv)

        lax.fori_loop(jnp.int32(0), jnp.int32(Tt + Ts - 1), walk_step, (_splat(0), _splat(0)),
                      unroll=False)

        def gamma_step(t, c):
            bvec = plsc.load_gather(best_v, [_splat(t)])
            for k in range(nk):
                gk = jnp.where(iot + k * _L >= bvec,
                               jnp.float32(1.0), jnp.float32(0.0))
                gam_v[t, pl.ds(k * _L, _L)] = gk
            return c

        lax.fori_loop(jnp.int32(0), jnp.int32(Tt), gamma_step, jnp.int32(0), unroll=False)

        def read_step(t, c):
            for k in range(nk):
                gk = gam_v[t + 1, pl.ds(k * _L, _L)]
                read_v[t, pl.ds(k * _L, _L)] = 1.0 - gk
            return c

        lax.fori_loop(jnp.int32(0), jnp.int32(Tt - 1), read_step, jnp.int32(0), unroll=False)

        pltpu.sync_copy(gam_v, gamma_hbm.at[wid])
        pltpu.sync_copy(read_v, read_hbm.at[wid])


def _sc_gamma(scores):
    B, Tt, Ts = scores.shape
    mesh = plsc.VectorSubcoreMesh(core_axis_name="c", subcore_axis_name="s")
    kern = pl.kernel(
        functools.partial(_sc_gamma_body, Tt, Ts),
        out_type=[
            jax.ShapeDtypeStruct((B, Tt, Ts), jnp.float32),
            jax.ShapeDtypeStruct((B, Tt - 1, Ts), jnp.float32),
        ],
        mesh=mesh,
        compiler_params=pltpu.CompilerParams(needs_layout_passes=False),
        scratch_types=[
            pltpu.VMEM((Tt, Ts), jnp.float32),
            pltpu.VMEM((2, Tt), jnp.float32),
            pltpu.VMEM((Tt + 1, Ts), jnp.float32),
            pltpu.VMEM((Tt + 1, Ts), jnp.float32),
            pltpu.VMEM((Tt,), jnp.int32),
            pltpu.VMEM((Tt, Ts), jnp.float32),
            pltpu.VMEM((Tt - 1, Ts), jnp.float32),
        ],
    )
    return kern(scores)


def _tc_gate_body(obs_ref, w_ref, b_ref, s_ref, sx_ref):
    x = jax.lax.dot_general(
        w_ref[...], obs_ref[0], (((1,), (1,)), ((), ())),
        preferred_element_type=jnp.float32,
    ) + b_ref[0, 0]
    s = jnp.minimum(x, 0.0) - jnp.log(1.0 + jnp.exp(-jnp.abs(x)))
    s_ref[0] = s
    sx_ref[0] = s - x


def _tc_gate(obs3, W, bc, rows_per_block):
    nblk, rows, Cdim = obs3.shape
    assert rows == rows_per_block
    return pl.pallas_call(
        _tc_gate_body,
        grid=(nblk,),
        in_specs=[
            pl.BlockSpec((1, rows, Cdim), lambda i: (i, _i0(), _i0())),
            pl.BlockSpec((1, Cdim), lambda i: (_i0(), _i0())),
            pl.BlockSpec((1, 128), lambda i: (_i0(), _i0())),
        ],
        out_specs=[
            pl.BlockSpec((1, 1, rows), lambda i: (i, _i0(), _i0())),
            pl.BlockSpec((1, 1, rows), lambda i: (i, _i0(), _i0())),
        ],
        out_shape=[
            jax.ShapeDtypeStruct((nblk, 1, rows), jnp.float32),
            jax.ShapeDtypeStruct((nblk, 1, rows), jnp.float32),
        ],
    )(obs3, W, bc)


def kernel(observations, scores, W, b):
    B, Tt, Ts, Cdim = observations.shape
    rows_per_block = 2048
    nblk = (B * Tt * Ts) // rows_per_block
    obs3 = observations.reshape(nblk, rows_per_block, Cdim)
    bc = jnp.broadcast_to(b.reshape(1, 1), (1, 128)).astype(jnp.float32)
    gamma, read = _sc_gamma(scores)
    s_flat, sx_flat = _tc_gate(obs3, W, bc, rows_per_block)
    s3 = s_flat.reshape(B, Tt, Ts)[:, : Tt - 1]
    sx3 = sx_flat.reshape(B, Tt, Ts)[:, : Tt - 1]
    controls = jnp.stack([s3, sx3], axis=-1)
    write = gamma[:, 1:]
    return (controls, gamma, read, write)


# speedup vs baseline: 1.4378x; 1.0206x over previous
import functools

import jax
import jax.numpy as jnp
from jax import lax
from jax.experimental import pallas as pl
from jax.experimental.pallas import tpu as pltpu
from jax.experimental.pallas import tpu_sc as plsc

_L = 16


def _i0():
    return jnp.int32(0)


def _iota16():
    return lax.iota(jnp.int32, 16)


def _splat(x):
    return jnp.full((_L,), x, dtype=jnp.int32)


def _cummin(v):
    return -plsc.cummax(-v)


def _rev(v):
    return lax.rev(v, (0,))


def _sc_gamma_body(Tt, Ts, scores_hbm, gamma_hbm, read_hbm, write_hbm,
                   sc_v, col_v, fs_v, cs_v, best_v, gam_v, read_v):
    B = scores_hbm.shape[0]
    NC = 1
    nk = Ts // _L
    wid = lax.axis_index("s") * NC + lax.axis_index("c")

    @pl.when(wid < B)
    def _worker():
        pltpu.sync_copy(scores_hbm.at[wid], sc_v)

        iot = _iota16()
        fiot = [(_iota16() + k * _L).astype(jnp.float32) for k in range(nk)]
        R = [fiot[k] * (fiot[k] + 3.0) * (0.5 / Ts) for k in range(nk)]
        C = [fiot[k] * (fiot[k] + 1.0) * (0.5 / Ts) for k in range(nk)]
        row0 = [(fiot[k] + 1.0) * (fiot[k] + 2.0) * (0.5 / Ts) for k in range(nk)]
        rowT = [(Ts - fiot[k]) * (1.0 / Ts) for k in range(nk)]

        zer = _splat(0)
        for col, slot in ((0, 0), (Ts - 1, 1)):
            carry = jnp.float32(0.0)
            for k in range(Tt // _L):
                rows = iot + k * _L
                g = plsc.load_gather(sc_v, [rows, _splat(col)])
                x = -g
                col_v[slot, pl.ds(k * _L, _L)] = plsc.cumsum(x) + carry
                carry = carry + jnp.sum(x)

        for k in range(nk):
            fs_v[0, pl.ds(k * _L, _L)] = row0[k]

        def fwd_step(t, fs_prev):
            c0 = plsc.load_gather(col_v, [zer, _splat(t)])
            u = []
            for k in range(nk):
                srow = sc_v[t, pl.ds(k * _L, _L)]
                uk = fs_prev[k] - srow - R[k]
                if k == 0:
                    uk = jnp.where(iot == 0, c0, uk)
                u.append(uk)
            out = []
            carry = jnp.float32(jnp.inf)
            for k in range(nk):
                run = jnp.minimum(_cummin(u[k]), carry)
                carry = run[_L - 1]
                fk = run + R[k]
                fs_v[t + 1, pl.ds(k * _L, _L)] = fk
                out.append(fk)
            return tuple(out)

        lax.fori_loop(jnp.int32(0), jnp.int32(Tt), fwd_step, tuple(row0), unroll=False)

        for k in range(nk):
            cs_v[Tt, pl.ds(k * _L, _L)] = fs_v[Tt, pl.ds(k * _L, _L)] + rowT[k]

        def bwd_step(i, bs_next):
            t = Tt - 1 - i
            cL = plsc.load_gather(col_v, [_splat(1), _splat(Tt - 1 - t)])
            w = []
            for k in range(nk):
                srow = sc_v[t, pl.ds(k * _L, _L)]
                wk = bs_next[k] - srow + C[k]
                if k == nk - 1:
                    wk = jnp.where(iot == _L - 1, cL + C[nk - 1], wk)
                w.append(wk)
            out = []
            carry = jnp.float32(jnp.inf)
            for k in range(nk - 1, -1, -1):
                sfx = jnp.minimum(_rev(_cummin(_rev(w[k]))), carry)
                carry = sfx[0]
                bk = sfx - C[k]
                fk = fs_v[t, pl.ds(k * _L, _L)]
                cs_v[t, pl.ds(k * _L, _L)] = bk + fk
                out.append(bk)
            out.reverse()
            return tuple(out)

        lax.fori_loop(jnp.int32(0), jnp.int32(Tt), bwd_step, tuple(rowT), unroll=False)

        for k in range(Tt // _L):
            best_v[pl.ds(k * _L, _L)] = _splat(Ts - 1)
        lane0 = iot == 0

        def walk_step(_, tj):
            tv, jv = tj
            active = (tv < Tt) & (jv < Ts - 1)
            tc = jnp.minimum(tv, Tt - 1)
            jc = jnp.minimum(jv, Ts - 2)
            g_dn = plsc.load_gather(cs_v, [tc + 1, jc])
            g_rt = plsc.load_gather(cs_v, [tc, jc + 1])
            do_write = active & (g_dn <= g_rt)
            do_read = active & jnp.logical_not(do_write)
            plsc.store_scatter(best_v, [tc], jc, mask=do_write & lane0)
            tv = jnp.where(do_write, tv + 1, tv)
            jv = jnp.where(do_read, jv + 1, jv)
            return (tv, jv)

        lax.fori_loop(jnp.int32(0), jnp.int32(Tt + Ts - 1), walk_step, (_splat(0), _splat(0)),
                      unroll=False)

        def gamma_step(t, c):
            bvec = plsc.load_gather(best_v, [_splat(t)])
            for k in range(nk):
                gk = jnp.where(iot + k * _L >= bvec,
                               jnp.float32(1.0), jnp.float32(0.0))
                gam_v[t, pl.ds(k * _L, _L)] = gk
            return c

        lax.fori_loop(jnp.int32(0), jnp.int32(Tt), gamma_step, jnp.int32(0), unroll=False)

        def read_step(t, c):
            for k in range(nk):
                gk = gam_v[t + 1, pl.ds(k * _L, _L)]
                read_v[t, pl.ds(k * _L, _L)] = 1.0 - gk
            return c

        lax.fori_loop(jnp.int32(0), jnp.int32(Tt - 1), read_step, jnp.int32(0), unroll=False)

        pltpu.sync_copy(gam_v, gamma_hbm.at[wid])
        pltpu.sync_copy(read_v, read_hbm.at[wid])
        pltpu.sync_copy(gam_v.at[pl.ds(1, Tt - 1)], write_hbm.at[wid])


def _sc_gamma(scores):
    B, Tt, Ts = scores.shape
    mesh = plsc.VectorSubcoreMesh(core_axis_name="c", subcore_axis_name="s",
                                  num_cores=1)
    kern = pl.kernel(
        functools.partial(_sc_gamma_body, Tt, Ts),
        out_type=[
            jax.ShapeDtypeStruct((B, Tt, Ts), jnp.float32),
            jax.ShapeDtypeStruct((B, Tt - 1, Ts), jnp.float32),
            jax.ShapeDtypeStruct((B, Tt - 1, Ts), jnp.float32),
        ],
        mesh=mesh,
        compiler_params=pltpu.CompilerParams(needs_layout_passes=False),
        scratch_types=[
            pltpu.VMEM((Tt, Ts), jnp.float32),
            pltpu.VMEM((2, Tt), jnp.float32),
            pltpu.VMEM((Tt + 1, Ts), jnp.float32),
            pltpu.VMEM((Tt + 1, Ts), jnp.float32),
            pltpu.VMEM((Tt,), jnp.int32),
            pltpu.VMEM((Tt, Ts), jnp.float32),
            pltpu.VMEM((Tt - 1, Ts), jnp.float32),
        ],
    )
    return kern(scores)


def _tc_gate_body(obs_ref, w_ref, b_ref, s_ref, sx_ref):
    x = jax.lax.dot_general(
        w_ref[...], obs_ref[0], (((1,), (1,)), ((), ())),
        preferred_element_type=jnp.float32,
    ) + b_ref[0, 0]
    s = jnp.minimum(x, 0.0) - jnp.log(1.0 + jnp.exp(-jnp.abs(x)))
    s_ref[0] = s
    sx_ref[0] = s - x


def _tc_gate(obs3, W, bc, rows_per_block):
    nblk, rows, Cdim = obs3.shape
    assert rows == rows_per_block
    return pl.pallas_call(
        _tc_gate_body,
        grid=(nblk,),
        in_specs=[
            pl.BlockSpec((1, rows, Cdim), lambda i: (i, _i0(), _i0())),
            pl.BlockSpec((1, Cdim), lambda i: (_i0(), _i0())),
            pl.BlockSpec((1, 128), lambda i: (_i0(), _i0())),
        ],
        out_specs=[
            pl.BlockSpec((1, 1, rows), lambda i: (i, _i0(), _i0())),
            pl.BlockSpec((1, 1, rows), lambda i: (i, _i0(), _i0())),
        ],
        out_shape=[
            jax.ShapeDtypeStruct((nblk, 1, rows), jnp.float32),
            jax.ShapeDtypeStruct((nblk, 1, rows), jnp.float32),
        ],
    )(obs3, W, bc)


def kernel(observations, scores, W, b):
    B, Tt, Ts, Cdim = observations.shape
    rows_per_block = 2048
    nblk = (B * Tt * Ts) // rows_per_block
    obs3 = observations.reshape(nblk, rows_per_block, Cdim)
    bc = jnp.broadcast_to(b.reshape(1, 1), (1, 128)).astype(jnp.float32)
    gamma, read, write = _sc_gamma(scores)
    s_flat, sx_flat = _tc_gate(obs3, W, bc, rows_per_block)
    s3 = s_flat.reshape(B, Tt, Ts)[:, : Tt - 1]
    sx3 = sx_flat.reshape(B, Tt, Ts)[:, : Tt - 1]
    controls = jnp.stack([s3, sx3], axis=-1)
    return (controls, gamma, read, write)


# speedup vs baseline: 1.4525x; 1.0102x over previous
import functools

import jax
import jax.numpy as jnp
from jax import lax
from jax.experimental import pallas as pl
from jax.experimental.pallas import tpu as pltpu
from jax.experimental.pallas import tpu_sc as plsc

_L = 16


def _i0():
    return jnp.int32(0)


def _iota16():
    return lax.iota(jnp.int32, 16)


def _splat(x):
    return jnp.full((_L,), x, dtype=jnp.int32)


def _cummin(v):
    return -plsc.cummax(-v)


def _rev(v):
    return lax.rev(v, (0,))


def _sc_gamma_body(Tt, Ts, scores_hbm, gamma_hbm, read_hbm, write_hbm,
                   sc_v, col_v, fs_v, cs_v, best_v, gam_v, read_v):
    B = scores_hbm.shape[0]
    NC = 1
    nk = Ts // _L
    wid = lax.axis_index("s") * NC + lax.axis_index("c")

    @pl.when(wid < B)
    def _worker():
        pltpu.sync_copy(scores_hbm.at[wid], sc_v)

        iot = _iota16()
        fiot = [(_iota16() + k * _L).astype(jnp.float32) for k in range(nk)]
        R = [fiot[k] * (fiot[k] + 3.0) * (0.5 / Ts) for k in range(nk)]
        C = [fiot[k] * (fiot[k] + 1.0) * (0.5 / Ts) for k in range(nk)]
        row0 = [(fiot[k] + 1.0) * (fiot[k] + 2.0) * (0.5 / Ts) for k in range(nk)]
        rowT = [(Ts - fiot[k]) * (1.0 / Ts) for k in range(nk)]

        zer = _splat(0)
        for col, slot in ((0, 0), (Ts - 1, 1)):
            carry = jnp.float32(0.0)
            for k in range(Tt // _L):
                rows = iot + k * _L
                g = plsc.load_gather(sc_v, [rows, _splat(col)])
                x = -g
                col_v[slot, pl.ds(k * _L, _L)] = plsc.cumsum(x) + carry
                carry = carry + jnp.sum(x)

        for k in range(nk):
            fs_v[0, pl.ds(k * _L, _L)] = row0[k]

        def fwd_step(t, fs_prev):
            c0 = plsc.load_gather(col_v, [zer, _splat(t)])
            u = []
            for k in range(nk):
                srow = sc_v[t, pl.ds(k * _L, _L)]
                uk = fs_prev[k] - srow - R[k]
                if k == 0:
                    uk = jnp.where(iot == 0, c0, uk)
                u.append(uk)
            out = []
            carry = jnp.float32(jnp.inf)
            for k in range(nk):
                run = jnp.minimum(_cummin(u[k]), carry)
                carry = run[_L - 1]
                fk = run + R[k]
                fs_v[t + 1, pl.ds(k * _L, _L)] = fk
                out.append(fk)
            return tuple(out)

        lax.fori_loop(jnp.int32(0), jnp.int32(Tt), fwd_step, tuple(row0), unroll=False)

        for k in range(nk):
            cs_v[Tt, pl.ds(k * _L, _L)] = fs_v[Tt, pl.ds(k * _L, _L)] + rowT[k]

        def bwd_step(i, bs_next):
            t = Tt - 1 - i
            cL = plsc.load_gather(col_v, [_splat(1), _splat(Tt - 1 - t)])
            w = []
            for k in range(nk):
                srow = sc_v[t, pl.ds(k * _L, _L)]
                wk = bs_next[k] - srow + C[k]
                if k == nk - 1:
                    wk = jnp.where(iot == _L - 1, cL + C[nk - 1], wk)
                w.append(wk)
            out = []
            carry = jnp.float32(jnp.inf)
            for k in range(nk - 1, -1, -1):
                sfx = jnp.minimum(_rev(_cummin(_rev(w[k]))), carry)
                carry = sfx[0]
                bk = sfx - C[k]
                fk = fs_v[t, pl.ds(k * _L, _L)]
                cs_v[t, pl.ds(k * _L, _L)] = bk + fk
                out.append(bk)
            out.reverse()
            return tuple(out)

        lax.fori_loop(jnp.int32(0), jnp.int32(Tt), bwd_step, tuple(rowT), unroll=False)

        for k in range(Tt // _L):
            best_v[pl.ds(k * _L, _L)] = _splat(Ts - 1)
        lane0 = iot == 0

        def walk_step(_, tj):
            tv, jv = tj
            active = (tv < Tt) & (jv < Ts - 1)
            tc = jnp.minimum(tv, Tt - 1)
            jc = jnp.minimum(jv, Ts - 2)
            g_dn = plsc.load_gather(cs_v, [tc + 1, jc])
            g_rt = plsc.load_gather(cs_v, [tc, jc + 1])
            do_write = active & (g_dn <= g_rt)
            do_read = active & jnp.logical_not(do_write)
            plsc.store_scatter(best_v, [tc], jc, mask=do_write & lane0)
            tv = jnp.where(do_write, tv + 1, tv)
            jv = jnp.where(do_read, jv + 1, jv)
            return (tv, jv)

        lax.fori_loop(jnp.int32(0), jnp.int32(Tt + Ts - 1), walk_step, (_splat(0), _splat(0)),
                      unroll=False)

        def gamma_step(t, c):
            bvec = plsc.load_gather(best_v, [_splat(t)])
            for k in range(nk):
                gk = jnp.where(iot + k * _L >= bvec,
                               jnp.float32(1.0), jnp.float32(0.0))
                gam_v[t, pl.ds(k * _L, _L)] = gk
            return c

        lax.fori_loop(jnp.int32(0), jnp.int32(Tt), gamma_step, jnp.int32(0), unroll=False)

        def read_step(t, c):
            for k in range(nk):
                gk = gam_v[t + 1, pl.ds(k * _L, _L)]
                read_v[t, pl.ds(k * _L, _L)] = 1.0 - gk
            return c

        lax.fori_loop(jnp.int32(0), jnp.int32(Tt - 1), read_step, jnp.int32(0), unroll=False)

        pltpu.sync_copy(gam_v, gamma_hbm.at[wid])
        pltpu.sync_copy(read_v, read_hbm.at[wid])
        pltpu.sync_copy(gam_v.at[pl.ds(1, Tt - 1)], write_hbm.at[wid])


def _sc_gamma(scores):
    B, Tt, Ts = scores.shape
    mesh = plsc.VectorSubcoreMesh(core_axis_name="c", subcore_axis_name="s",
                                  num_cores=1)
    kern = pl.kernel(
        functools.partial(_sc_gamma_body, Tt, Ts),
        out_type=[
            jax.ShapeDtypeStruct((B, Tt, Ts), jnp.float32),
            jax.ShapeDtypeStruct((B, Tt - 1, Ts), jnp.float32),
            jax.ShapeDtypeStruct((B, Tt - 1, Ts), jnp.float32),
        ],
        mesh=mesh,
        compiler_params=pltpu.CompilerParams(needs_layout_passes=False),
        scratch_types=[
            pltpu.VMEM((Tt, Ts), jnp.float32),
            pltpu.VMEM((2, Tt), jnp.float32),
            pltpu.VMEM((Tt + 1, Ts), jnp.float32),
            pltpu.VMEM((Tt + 1, Ts), jnp.float32),
            pltpu.VMEM((Tt,), jnp.int32),
            pltpu.VMEM((Tt, Ts), jnp.float32),
            pltpu.VMEM((Tt - 1, Ts), jnp.float32),
        ],
    )
    return kern(scores)


def _tc_gate_body(obs_ref, w_ref, b_ref, s_ref, sx_ref):
    for g in range(obs_ref.shape[1]):
        x = jax.lax.dot_general(
            w_ref[...], obs_ref[0, g], (((1,), (1,)), ((), ())),
            preferred_element_type=jnp.float32,
        ) + b_ref[0, 0]
        s = jnp.minimum(x, 0.0) - jnp.log(1.0 + jnp.exp(-jnp.abs(x)))
        s_ref[0, pl.ds(g, 1)] = s
        sx_ref[0, pl.ds(g, 1)] = s - x


def _tc_gate(obs4, W, bc):
    nblk, ng, rows, Cdim = obs4.shape
    return pl.pallas_call(
        _tc_gate_body,
        grid=(nblk,),
        in_specs=[
            pl.BlockSpec((1, ng, rows, Cdim), lambda i: (i, _i0(), _i0(), _i0())),
            pl.BlockSpec((1, Cdim), lambda i: (_i0(), _i0())),
            pl.BlockSpec((1, 128), lambda i: (_i0(), _i0())),
        ],
        out_specs=[
            pl.BlockSpec((1, ng, rows), lambda i: (i, _i0(), _i0())),
            pl.BlockSpec((1, ng, rows), lambda i: (i, _i0(), _i0())),
        ],
        out_shape=[
            jax.ShapeDtypeStruct((nblk, ng, rows), jnp.float32),
            jax.ShapeDtypeStruct((nblk, ng, rows), jnp.float32),
        ],
    )(obs4, W, bc)


def kernel(observations, scores, W, b):
    B, Tt, Ts, Cdim = observations.shape
    rows_per_block = 2048
    nblk = (B * Tt * Ts) // rows_per_block
    obs4 = observations.reshape(nblk, 8, rows_per_block // 8, Cdim)
    bc = jnp.broadcast_to(b.reshape(1, 1), (1, 128)).astype(jnp.float32)
    gamma, read, write = _sc_gamma(scores)
    s_flat, sx_flat = _tc_gate(obs4, W, bc)
    s3 = s_flat.reshape(B, Tt, Ts)[:, : Tt - 1]
    sx3 = sx_flat.reshape(B, Tt, Ts)[:, : Tt - 1]
    controls = jnp.stack([s3, sx3], axis=-1)
    return (controls, gamma, read, write)
